# baseline (speedup 1.0000x reference)
import functools

import jax
import jax.numpy as jnp
from jax import lax
from jax.experimental import pallas as pl
from jax.experimental.pallas import tpu as pltpu
from jax.experimental.pallas import tpu_sc as plsc

N_NODES = 10000
IN_CH = 128
OUT_CH = 16
HEADS = 8
D = HEADS * OUT_CH
TW = D + 8
NC = 2
NS = 16
NW = NC * NS
NPAD = 10240
RPT = NPAD // NS
NEG_SLOPE = 0.2
CH = 64
NBUF = 4


def _prep_body(x_ref, w_ref, as_ref, ad_ref, t1_ref, t2_ref, m_ref):
    xp = jnp.dot(x_ref[...], w_ref[...], preferred_element_type=jnp.float32)
    jj = lax.broadcasted_iota(jnp.int32, (D, 16), 0)
    hh = lax.broadcasted_iota(jnp.int32, (D, 16), 1)
    sel = ((jj // OUT_CH) == hh).astype(jnp.float32)
    asrc = jnp.dot(xp * as_ref[...], sel, preferred_element_type=jnp.float32)
    adst = jnp.dot(xp * ad_ref[...], sel, preferred_element_type=jnp.float32)
    t1_ref[:, :D] = xp
    t1_ref[:, D:] = asrc[:, :8]
    t2_ref[...] = adst
    s = (jnp.max(asrc, axis=0, keepdims=True)
         + jnp.max(adst, axis=0, keepdims=True))
    m_ref[...] = jnp.maximum(s, NEG_SLOPE * s)


def _finish_body(acc_ref, t1_ref, t2_ref, m_ref, b_ref, o_ref):
    accs = acc_ref[0, :N_NODES, :] + acc_ref[1, :N_NODES, :]
    xp = t1_ref[:, :D]
    s = t1_ref[:, D:] + t2_ref[:, :8]
    s = jnp.maximum(s, NEG_SLOPE * s)
    fs = jnp.exp(s - m_ref[:, :8])
    hh = lax.broadcasted_iota(jnp.int32, (8, D), 0)
    jj = lax.broadcasted_iota(jnp.int32, (8, D), 1)
    sel = (hh == (jj // OUT_CH)).astype(jnp.float32)
    fs128 = jnp.dot(fs, sel, preferred_element_type=jnp.float32)
    den128 = jnp.dot(accs[:, D:] + fs, sel, preferred_element_type=jnp.float32)
    num = accs[:, :D] + fs128 * xp
    out = num / (den128 + 1e-16) + b_ref[...]
    o_ref[...] = jnp.maximum(out, 0.0)


def _bcast_lane(v, h):
    idx = jnp.full((16, 1), h, jnp.int32)
    return lax.gather(
        v, idx,
        lax.GatherDimensionNumbers(offset_dims=(), collapsed_slice_dims=(0,),
                                   start_index_map=(0,)),
        slice_sizes=(1,),
        mode=lax.GatherScatterMode.PROMISE_IN_BOUNDS)


def _edge_pass(t1, t2, src, dst, m, zrows, n_edges):
    ept = n_edges // NW
    nch = ept // CH
    cht = ept % CH
    ngrp = nch // 2
    assert nch % 2 == 0 and cht % 16 == 0
    mesh = plsc.VectorSubcoreMesh(core_axis_name="c", subcore_axis_name="s")

    @functools.partial(
        pl.kernel,
        out_type=jax.ShapeDtypeStruct((NC, NPAD, TW), jnp.float32),
        mesh=mesh,
        scratch_types=[
            pltpu.VMEM((NBUF, CH), jnp.int32),
            pltpu.VMEM((NBUF, CH), jnp.int32),
            pltpu.VMEM((NBUF, CH, TW), jnp.float32),
            pltpu.VMEM((NBUF, CH, 16), jnp.float32),
            pltpu.VMEM((1, 16), jnp.float32),
            pltpu.VMEM((16,), jnp.int32),
            pltpu.VMEM((16,), jnp.int32),
            pltpu.VMEM_SHARED((NPAD, TW), jnp.float32),
            pltpu.SemaphoreType.DMA((NBUF,)),
            pltpu.SemaphoreType.DMA((NBUF,)),
            pltpu.SemaphoreType.DMA((NBUF,)),
            pltpu.SemaphoreType.DMA((NBUF,)),
        ],
        compiler_params=pltpu.CompilerParams(needs_layout_passes=False,
                                             use_tc_tiling_on_sc=False),
    )
    def ek(t1_hbm, t2_hbm, src_hbm, dst_hbm, m_hbm, z_hbm, out_hbm,
           src_v, dst_v, rows_v, adst_v, m_v, tsrc_v, tdst_v, acc_sh,
           sem_i, sem_r, sem_a, sem_s):
        cid = lax.axis_index("c")
        sid = lax.axis_index("s")
        wid = cid * NS + sid
        pltpu.sync_copy(z_hbm, acc_sh.at[pl.ds(sid * RPT, RPT)])
        pltpu.sync_copy(m_hbm, m_v)
        plsc.subcore_barrier()
        ebase = wid * ept
        lanes = lax.iota(jnp.int32, 16)

        mvec = m_v[0, :]
        mhs = [_bcast_lane(mvec, h) for h in range(HEADS)]

        def compute_edges(rows, adst, n):
            for h in range(HEADS):
                mh = mhs[h]
                colf = jnp.full((16,), D + h, jnp.int32)
                colb = jnp.full((16,), h, jnp.int32)
                for g in range(n // 16):
                    eids = lanes + g * 16
                    av = plsc.load_gather(rows, [eids, colf])
                    bv = plsc.load_gather(adst, [eids, colb])
                    e = av + bv
                    e = jnp.maximum(e, NEG_SLOPE * e)
                    plsc.store_scatter(rows, [eids, colf], jnp.exp(e - mh))

            def scale(i, carry2):
                for u in range(2):
                    e = i * 2 + u
                    fv = rows[e, pl.ds(TW - 16, 16)]
                    for h in range(HEADS):
                        fb = _bcast_lane(fv, h + 8)
                        blk = rows[e, pl.ds(h * OUT_CH, OUT_CH)]
                        rows[e, pl.ds(h * OUT_CH, OUT_CH)] = blk * fb
                return carry2
            lax.fori_loop(0, n // 2, scale, 0)

        def fire_idx(b, base, scatter_wait):
            if scatter_wait:
                pltpu.make_async_copy(rows_v.at[b], acc_sh.at[dst_v.at[b]],
                                      sem_s.at[b]).wait()
            pltpu.async_copy(src_hbm.at[pl.ds(base, CH)],
                             src_v.at[b], sem_i.at[b])
            pltpu.async_copy(dst_hbm.at[pl.ds(base, CH)],
                             dst_v.at[b], sem_i.at[b])

        def fire_rows(b, base):
            pltpu.make_async_copy(src_hbm.at[pl.ds(base, CH)],
                                  src_v.at[b], sem_i.at[b]).wait()
            pltpu.make_async_copy(dst_hbm.at[pl.ds(base, CH)],
                                  dst_v.at[b], sem_i.at[b]).wait()
            pltpu.async_copy(t1_hbm.at[src_v.at[b]], rows_v.at[b],
                             sem_r.at[b])
            pltpu.async_copy(t2_hbm.at[dst_v.at[b]], adst_v.at[b],
                             sem_a.at[b])

        def consume_buf(b):
            pltpu.make_async_copy(t1_hbm.at[src_v.at[b]],
                                  rows_v.at[b], sem_r.at[b]).wait()
            pltpu.make_async_copy(t2_hbm.at[dst_v.at[b]],
                                  adst_v.at[b], sem_a.at[b]).wait()
            compute_edges(rows_v.at[b], adst_v.at[b], CH)
            pltpu.async_copy(rows_v.at[b], acc_sh.at[dst_v.at[b]],
                             sem_s.at[b], add=True)

        def fire_idx_k(gidx, k, scatter_wait):
            base = ebase + (gidx * 2 + k) * CH

            @pl.when(gidx % 2 == 0)
            def _():
                fire_idx(0 + k, base, scatter_wait)

            @pl.when(gidx % 2 == 1)
            def _():
                fire_idx(2 + k, base, scatter_wait)

        def fire_rows_k(gidx, k):
            base = ebase + (gidx * 2 + k) * CH

            @pl.when(gidx % 2 == 0)
            def _():
                fire_rows(0 + k, base)

            @pl.when(gidx % 2 == 1)
            def _():
                fire_rows(2 + k, base)

        def consume_k(gidx, k):
            @pl.when(gidx % 2 == 0)
            def _():
                consume_buf(0 + k)

            @pl.when(gidx % 2 == 1)
            def _():
                consume_buf(2 + k)

        for k in range(2):
            fire_idx_k(jnp.int32(0), k, scatter_wait=False)
        for k in range(2):
            fire_rows_k(jnp.int32(0), k)

        def group(g, carry):
            @pl.when(g + 1 < ngrp)
            def _():
                @pl.when(g >= 1)
                def _():
                    fire_idx_k(g + 1, 0, scatter_wait=True)

                @pl.when(g < 1)
                def _():
                    fire_idx_k(g + 1, 0, scatter_wait=False)
            consume_k(g, 0)

            @pl.when(g + 1 < ngrp)
            def _():
                @pl.when(g >= 1)
                def _():
                    fire_idx_k(g + 1, 1, scatter_wait=True)

                @pl.when(g < 1)
                def _():
                    fire_idx_k(g + 1, 1, scatter_wait=False)

                fire_rows_k(g + 1, 0)
            consume_k(g, 1)

            @pl.when(g + 1 < ngrp)
            def _():
                fire_rows_k(g + 1, 1)
            return carry

        lax.fori_loop(0, ngrp, group, 0)
        for b in range(NBUF):
            pltpu.make_async_copy(rows_v.at[b], acc_sh.at[dst_v.at[b]],
                                  sem_s.at[b]).wait()

        if cht:
            tbase = ebase + nch * CH
            trows = rows_v.at[0, pl.ds(0, cht)]
            tadst = adst_v.at[0, pl.ds(0, cht)]
            pltpu.sync_copy(src_hbm.at[pl.ds(tbase, cht)], tsrc_v)
            pltpu.sync_copy(dst_hbm.at[pl.ds(tbase, cht)], tdst_v)
            pltpu.async_copy(t1_hbm.at[tsrc_v], trows, sem_r.at[0]).wait()
            pltpu.async_copy(t2_hbm.at[tdst_v], tadst, sem_a.at[0]).wait()
            compute_edges(trows, tadst, cht)
            pltpu.sync_copy(trows, acc_sh.at[tdst_v], add=True)

        plsc.subcore_barrier()
        pltpu.sync_copy(acc_sh.at[pl.ds(sid * RPT, RPT)],
                        out_hbm.at[cid, pl.ds(sid * RPT, RPT)])

    return ek(t1, t2, src, dst, m, zrows)


def kernel(x, edge_index, W, att_src, att_dst, bias):
    src = edge_index[0].astype(jnp.int32)
    dst = edge_index[1].astype(jnp.int32)
    n_edges = src.shape[0]
    att_s = att_src.reshape(1, D)
    att_d = att_dst.reshape(1, D)

    t1, t2, m = pl.pallas_call(
        _prep_body,
        out_shape=[
            jax.ShapeDtypeStruct((N_NODES, TW), jnp.float32),
            jax.ShapeDtypeStruct((N_NODES, 16), jnp.float32),
            jax.ShapeDtypeStruct((1, 16), jnp.float32),
        ],
    )(x, W, att_s, att_d)

    zrows = jnp.zeros((RPT, TW), jnp.float32)
    acc = _edge_pass(t1, t2, src, dst, m, zrows, n_edges)

    out = pl.pallas_call(
        _finish_body,
        out_shape=jax.ShapeDtypeStruct((N_NODES, D), jnp.float32),
    )(acc, t1, t2, m, bias.reshape(1, D))
    return out


# speedup vs baseline: 1.0256x; 1.0256x over previous
import functools

import jax
import jax.numpy as jnp
from jax import lax
from jax.experimental import pallas as pl
from jax.experimental.pallas import tpu as pltpu
from jax.experimental.pallas import tpu_sc as plsc

N_NODES = 10000
IN_CH = 128
OUT_CH = 16
HEADS = 8
D = HEADS * OUT_CH
TW = D + 8
NC = 2
NS = 16
NW = NC * NS
NPAD = 10240
RPT = NPAD // NS
NEG_SLOPE = 0.2
CH = 64
NBUF = 4


def _prep_body(x_ref, w_ref, as_ref, ad_ref, t1_ref, t2_ref, m_ref):
    xp = jnp.dot(x_ref[...], w_ref[...], preferred_element_type=jnp.float32)
    jj = lax.broadcasted_iota(jnp.int32, (D, 16), 0)
    hh = lax.broadcasted_iota(jnp.int32, (D, 16), 1)
    sel = ((jj // OUT_CH) == hh).astype(jnp.float32)
    asrc = jnp.dot(xp * as_ref[...], sel, preferred_element_type=jnp.float32)
    adst = jnp.dot(xp * ad_ref[...], sel, preferred_element_type=jnp.float32)
    t1_ref[:, :D] = xp
    t1_ref[:, D:] = asrc[:, :8]
    t2_ref[...] = adst
    s = (jnp.max(asrc, axis=0, keepdims=True)
         + jnp.max(adst, axis=0, keepdims=True))
    m_ref[...] = jnp.maximum(s, NEG_SLOPE * s)


def _finish_body(acc_ref, t1_ref, t2_ref, m_ref, b_ref, o_ref):
    accs = acc_ref[0, :N_NODES, :] + acc_ref[1, :N_NODES, :]
    xp = t1_ref[:, :D]
    s = t1_ref[:, D:] + t2_ref[:, :8]
    s = jnp.maximum(s, NEG_SLOPE * s)
    fs = jnp.exp(s - m_ref[:, :8])
    hh = lax.broadcasted_iota(jnp.int32, (8, D), 0)
    jj = lax.broadcasted_iota(jnp.int32, (8, D), 1)
    sel = (hh == (jj // OUT_CH)).astype(jnp.float32)
    fs128 = jnp.dot(fs, sel, preferred_element_type=jnp.float32)
    den128 = jnp.dot(accs[:, D:] + fs, sel, preferred_element_type=jnp.float32)
    num = accs[:, :D] + fs128 * xp
    out = num / (den128 + 1e-16) + b_ref[...]
    o_ref[...] = jnp.maximum(out, 0.0)


def _bcast_lane(v, h):
    idx = jnp.full((16, 1), h, jnp.int32)
    return lax.gather(
        v, idx,
        lax.GatherDimensionNumbers(offset_dims=(), collapsed_slice_dims=(0,),
                                   start_index_map=(0,)),
        slice_sizes=(1,),
        mode=lax.GatherScatterMode.PROMISE_IN_BOUNDS)


def _edge_pass(t1, t2, src, dst, m, zrows, n_edges):
    ept = n_edges // NW
    nch = ept // CH
    cht = ept % CH
    ngrp = nch // 2
    assert nch % 2 == 0 and cht % 16 == 0
    mesh = plsc.VectorSubcoreMesh(core_axis_name="c", subcore_axis_name="s")

    @functools.partial(
        pl.kernel,
        out_type=jax.ShapeDtypeStruct((NC, NPAD, TW), jnp.float32),
        mesh=mesh,
        scratch_types=[
            pltpu.VMEM((NBUF, CH), jnp.int32),
            pltpu.VMEM((NBUF, CH), jnp.int32),
            pltpu.VMEM((NBUF, CH, TW), jnp.float32),
            pltpu.VMEM((NBUF, CH, 16), jnp.float32),
            pltpu.VMEM((1, 16), jnp.float32),
            pltpu.VMEM((16,), jnp.int32),
            pltpu.VMEM((16,), jnp.int32),
            pltpu.VMEM_SHARED((NPAD, TW), jnp.float32),
            pltpu.SemaphoreType.DMA((NBUF,)),
            pltpu.SemaphoreType.DMA((NBUF,)),
            pltpu.SemaphoreType.DMA((NBUF,)),
            pltpu.SemaphoreType.DMA((NBUF,)),
        ],
        compiler_params=pltpu.CompilerParams(needs_layout_passes=False,
                                             use_tc_tiling_on_sc=False),
    )
    def ek(t1_hbm, t2_hbm, src_hbm, dst_hbm, m_hbm, z_hbm, out_hbm,
           src_v, dst_v, rows_v, adst_v, m_v, tsrc_v, tdst_v, acc_sh,
           sem_i, sem_r, sem_a, sem_s):
        cid = lax.axis_index("c")
        sid = lax.axis_index("s")
        wid = cid * NS + sid
        pltpu.sync_copy(z_hbm, acc_sh.at[pl.ds(sid * RPT, RPT)])
        pltpu.sync_copy(m_hbm, m_v)
        plsc.subcore_barrier()
        ebase = wid * ept
        lanes = lax.iota(jnp.int32, 16)

        def compute_edges(rows, adst, n):
            mvec = m_v[0, :]
            for h in range(HEADS):
                mh = _bcast_lane(mvec, h)
                colf = jnp.full((16,), D + h, jnp.int32)
                colb = jnp.full((16,), h, jnp.int32)
                for g in range(n // 16):
                    eids = lanes + g * 16
                    av = plsc.load_gather(rows, [eids, colf])
                    bv = plsc.load_gather(adst, [eids, colb])
                    e = av + bv
                    e = jnp.maximum(e, NEG_SLOPE * e)
                    plsc.store_scatter(rows, [eids, colf], jnp.exp(e - mh))

            def scale(i, carry2):
                for u in range(2):
                    e = i * 2 + u
                    fv = rows[e, pl.ds(TW - 16, 16)]
                    for h in range(HEADS):
                        fb = _bcast_lane(fv, h + 8)
                        blk = rows[e, pl.ds(h * OUT_CH, OUT_CH)]
                        rows[e, pl.ds(h * OUT_CH, OUT_CH)] = blk * fb
                return carry2
            lax.fori_loop(0, n // 2, scale, 0)

        def fire_idx(b, base, scatter_wait):
            if scatter_wait:
                pltpu.make_async_copy(rows_v.at[b], acc_sh.at[dst_v.at[b]],
                                      sem_s.at[b]).wait()
            pltpu.async_copy(src_hbm.at[pl.ds(base, CH)],
                             src_v.at[b], sem_i.at[b])
            pltpu.async_copy(dst_hbm.at[pl.ds(base, CH)],
                             dst_v.at[b], sem_i.at[b])

        def fire_rows(b, base):
            pltpu.make_async_copy(src_hbm.at[pl.ds(base, CH)],
                                  src_v.at[b], sem_i.at[b]).wait()
            pltpu.make_async_copy(dst_hbm.at[pl.ds(base, CH)],
                                  dst_v.at[b], sem_i.at[b]).wait()
            pltpu.async_copy(t1_hbm.at[src_v.at[b]], rows_v.at[b],
                             sem_r.at[b])
            pltpu.async_copy(t2_hbm.at[dst_v.at[b]], adst_v.at[b],
                             sem_a.at[b])

        def consume_buf(b):
            pltpu.make_async_copy(t1_hbm.at[src_v.at[b]],
                                  rows_v.at[b], sem_r.at[b]).wait()
            pltpu.make_async_copy(t2_hbm.at[dst_v.at[b]],
                                  adst_v.at[b], sem_a.at[b]).wait()
            compute_edges(rows_v.at[b], adst_v.at[b], CH)
            pltpu.async_copy(rows_v.at[b], acc_sh.at[dst_v.at[b]],
                             sem_s.at[b], add=True)

        def fire_idx_k(gidx, k, scatter_wait):
            base = ebase + (gidx * 2 + k) * CH

            @pl.when(gidx % 2 == 0)
            def _():
                fire_idx(0 + k, base, scatter_wait)

            @pl.when(gidx % 2 == 1)
            def _():
                fire_idx(2 + k, base, scatter_wait)

        def fire_rows_k(gidx, k):
            base = ebase + (gidx * 2 + k) * CH

            @pl.when(gidx % 2 == 0)
            def _():
                fire_rows(0 + k, base)

            @pl.when(gidx % 2 == 1)
            def _():
                fire_rows(2 + k, base)

        def consume_k(gidx, k):
            @pl.when(gidx % 2 == 0)
            def _():
                consume_buf(0 + k)

            @pl.when(gidx % 2 == 1)
            def _():
                consume_buf(2 + k)

        for k in range(2):
            fire_idx_k(jnp.int32(0), k, scatter_wait=False)
        for k in range(2):
            fire_rows_k(jnp.int32(0), k)

        def group(g, carry):
            @pl.when(g + 1 < ngrp)
            def _():
                @pl.when(g >= 1)
                def _():
                    fire_idx_k(g + 1, 0, scatter_wait=True)

                @pl.when(g < 1)
                def _():
                    fire_idx_k(g + 1, 0, scatter_wait=False)
            consume_k(g, 0)

            @pl.when(g + 1 < ngrp)
            def _():
                @pl.when(g >= 1)
                def _():
                    fire_idx_k(g + 1, 1, scatter_wait=True)

                @pl.when(g < 1)
                def _():
                    fire_idx_k(g + 1, 1, scatter_wait=False)

                fire_rows_k(g + 1, 0)
            consume_k(g, 1)

            @pl.when(g + 1 < ngrp)
            def _():
                fire_rows_k(g + 1, 1)
            return carry

        lax.fori_loop(0, ngrp, group, 0)
        for b in range(NBUF):
            pltpu.make_async_copy(rows_v.at[b], acc_sh.at[dst_v.at[b]],
                                  sem_s.at[b]).wait()

        if cht:
            tbase = ebase + nch * CH
            trows = rows_v.at[0, pl.ds(0, cht)]
            tadst = adst_v.at[0, pl.ds(0, cht)]
            pltpu.sync_copy(src_hbm.at[pl.ds(tbase, cht)], tsrc_v)
            pltpu.sync_copy(dst_hbm.at[pl.ds(tbase, cht)], tdst_v)
            pltpu.async_copy(t1_hbm.at[tsrc_v], trows, sem_r.at[0]).wait()
            pltpu.async_copy(t2_hbm.at[tdst_v], tadst, sem_a.at[0]).wait()
            compute_edges(trows, tadst, cht)
            pltpu.sync_copy(trows, acc_sh.at[tdst_v], add=True)

        plsc.subcore_barrier()
        pltpu.sync_copy(acc_sh.at[pl.ds(sid * RPT, RPT)],
                        out_hbm.at[cid, pl.ds(sid * RPT, RPT)])

    return ek(t1, t2, src, dst, m, zrows)


def kernel(x, edge_index, W, att_src, att_dst, bias):
    src = edge_index[0].astype(jnp.int32)
    dst = edge_index[1].astype(jnp.int32)
    n_edges = src.shape[0]
    att_s = att_src.reshape(1, D)
    att_d = att_dst.reshape(1, D)

    t1, t2, m = pl.pallas_call(
        _prep_body,
        out_shape=[
            jax.ShapeDtypeStruct((N_NODES, TW), jnp.float32),
            jax.ShapeDtypeStruct((N_NODES, 16), jnp.float32),
            jax.ShapeDtypeStruct((1, 16), jnp.float32),
        ],
    )(x, W, att_s, att_d)

    zrows = jnp.zeros((RPT, TW), jnp.float32)
    acc = _edge_pass(t1, t2, src, dst, m, zrows, n_edges)

    out = pl.pallas_call(
        _finish_body,
        out_shape=jax.ShapeDtypeStruct((N_NODES, D), jnp.float32),
    )(acc, t1, t2, m, bias.reshape(1, D))
    return out
